# baseline (speedup 1.0000x reference)
import jax
import jax.numpy as jnp
from jax.experimental import pallas as pl
from jax.experimental.pallas import tpu as pltpu

B, T = 64, 20
C, R = 2048, 49
RP = 56
H, E, VOCAB = 512, 256, 32000
F32 = jnp.float32

_EBB = 8
_EROWS = _EBB * RP


def _enc_body(x_ref, wa_ref, ba_ref, wb_ref, bb_ref, wv_ref,
              v_ref, vp_ref, vg_ref):
    x = x_ref[...]
    v2 = jnp.dot(x, wa_ref[...], preferred_element_type=F32) + ba_ref[...]
    v2 = jnp.maximum(v2, 0.0)
    m = jax.lax.broadcasted_iota(jnp.int32, (_EBB, _EROWS), 1)
    bidx = jax.lax.broadcasted_iota(jnp.int32, (_EBB, _EROWS), 0)
    sel = jnp.where(m // RP == bidx, 1.0 / R, 0.0).astype(F32)
    a_g = jnp.dot(sel, x, preferred_element_type=F32)
    vg = jnp.dot(a_g, wb_ref[...], preferred_element_type=F32) + bb_ref[...]
    vg_ref[...] = jnp.maximum(vg, 0.0)
    vp2 = jnp.dot(v2, wv_ref[...], preferred_element_type=F32)
    v_ref[...] = v2.reshape(_EBB, RP, H)
    vp_ref[...] = vp2.reshape(_EBB, RP, RP)


def _encoder(x2, wa_t, ba2, wb_t, bb2, wv_t):
    n = B // _EBB
    return pl.pallas_call(
        _enc_body,
        grid=(n,),
        in_specs=[
            pl.BlockSpec((_EROWS, C), lambda i: (i, 0)),
            pl.BlockSpec((C, H), lambda i: (0, 0)),
            pl.BlockSpec((1, H), lambda i: (0, 0)),
            pl.BlockSpec((C, H), lambda i: (0, 0)),
            pl.BlockSpec((1, H), lambda i: (0, 0)),
            pl.BlockSpec((H, RP), lambda i: (0, 0)),
        ],
        out_specs=[
            pl.BlockSpec((_EBB, RP, H), lambda i: (i, 0, 0)),
            pl.BlockSpec((_EBB, RP, RP), lambda i: (i, 0, 0)),
            pl.BlockSpec((_EBB, H), lambda i: (i, 0)),
        ],
        out_shape=[
            jax.ShapeDtypeStruct((B, RP, H), F32),
            jax.ShapeDtypeStruct((B, RP, RP), F32),
            jax.ShapeDtypeStruct((B, H), F32),
        ],
        compiler_params=pltpu.CompilerParams(
            dimension_semantics=("parallel",),
            vmem_limit_bytes=100 * 1024 * 1024,
        ),
    )(x2, wa_t, ba2, wb_t, bb2, wv_t)


_BH = B // 2


def _dec_body(v_ref, vp_ref, vg_ref, wg_ref, wh_ref, wcat_ref, bg_ref,
              cap_ref, emb_hbm, hid_ref, ebuf, sem):
    b0 = pl.program_id(0) * _BH

    def fetch(t, slot):
        for b in range(_BH):
            idx = cap_ref[b0 + b, t]
            pltpu.make_async_copy(
                emb_hbm.at[pl.ds(idx, 1)],
                ebuf.at[slot, pl.ds(b, 1)],
                sem.at[slot],
            ).start()

    fetch(0, 0)
    h = vg_ref[...]
    c = h
    wh_v = wh_ref[...]
    rmask = jax.lax.broadcasted_iota(jnp.int32, (_BH, RP), 1) < R

    for t in range(T):
        slot = t % 2
        if t + 1 < T:
            fetch(t + 1, (t + 1) % 2)
        pltpu.make_async_copy(ebuf.at[slot], ebuf.at[slot], sem.at[slot]).wait()
        e_t = ebuf[slot]
        hwg = jnp.dot(h, wg_ref[...], preferred_element_type=F32)
        content = vp_ref[...] + hwg[:, None, :]
        z = jnp.sum(jnp.tanh(content) * wh_v, axis=2)
        z = jnp.where(rmask, z, -1e30)
        ez = jnp.exp(z - jnp.max(z, axis=1, keepdims=True))
        alpha = ez / jnp.sum(ez, axis=1, keepdims=True)
        c_att = jnp.sum(alpha[:, :, None] * v_ref[...], axis=1)
        xcat = jnp.concatenate([c_att, e_t, h], axis=1)
        gates = jnp.dot(xcat, wcat_ref[...],
                        preferred_element_type=F32) + bg_ref[...]
        ii = jax.nn.sigmoid(gates[:, 0:H])
        ff = jax.nn.sigmoid(gates[:, H:2 * H])
        gg = jnp.tanh(gates[:, 2 * H:3 * H])
        oo = jax.nn.sigmoid(gates[:, 3 * H:4 * H])
        c = ff * c + ii * gg
        h = oo * jnp.tanh(c)
        hid_ref[t] = jnp.concatenate([c_att, h], axis=1)


def _decode(v3, vp3, vg, wg_t, wh3, wcat, bg2, cap32, embed):
    return pl.pallas_call(
        _dec_body,
        grid=(2,),
        in_specs=[
            pl.BlockSpec((_BH, RP, H), lambda i: (i, 0, 0)),
            pl.BlockSpec((_BH, RP, RP), lambda i: (i, 0, 0)),
            pl.BlockSpec((_BH, H), lambda i: (i, 0)),
            pl.BlockSpec((H, RP), lambda i: (0, 0)),
            pl.BlockSpec((1, 1, RP), lambda i: (0, 0, 0)),
            pl.BlockSpec((H + E + H, 4 * H), lambda i: (0, 0)),
            pl.BlockSpec((1, 4 * H), lambda i: (0, 0)),
            pl.BlockSpec(memory_space=pltpu.SMEM),
            pl.BlockSpec(memory_space=pl.ANY),
        ],
        out_specs=pl.BlockSpec((T, _BH, 2 * H), lambda i: (0, i, 0)),
        out_shape=jax.ShapeDtypeStruct((T, B, 2 * H), F32),
        scratch_shapes=[
            pltpu.VMEM((2, _BH, E), F32),
            pltpu.SemaphoreType.DMA((2,)),
        ],
        compiler_params=pltpu.CompilerParams(
            dimension_semantics=("parallel",),
            vmem_limit_bytes=100 * 1024 * 1024,
        ),
    )(v3, vp3, vg, wg_t, wh3, wcat, bg2, cap32, embed)


_VB = 1280


def _mlp_body(x_ref, w_ref, b_ref, o_ref):
    w = w_ref[...].astype(jnp.bfloat16)
    s = jax.lax.dot_general(x_ref[...], w, (((1,), (1,)), ((), ())),
                            preferred_element_type=F32)
    o_ref[...] = s + b_ref[...]


def _mlp(xb, w_mlp, bm2):
    n = VOCAB // _VB
    return pl.pallas_call(
        _mlp_body,
        grid=(n,),
        in_specs=[
            pl.BlockSpec((B * T, 2 * H), lambda i: (0, 0)),
            pl.BlockSpec((_VB, 2 * H), lambda i: (i, 0)),
            pl.BlockSpec((1, _VB), lambda i: (0, i)),
        ],
        out_specs=pl.BlockSpec((B * T, _VB), lambda i: (0, i)),
        out_shape=jax.ShapeDtypeStruct((B * T, VOCAB), F32),
        compiler_params=pltpu.CompilerParams(
            dimension_semantics=("parallel",),
            vmem_limit_bytes=100 * 1024 * 1024,
        ),
    )(xb, w_mlp, bm2)


def kernel(feat, captions, lengths, W_a, b_a, W_b, b_b, embed,
           Wv, Wg, Wh, W_ih, W_hh, b_ih, b_hh, W_mlp, b_mlp):
    vmap3 = feat.reshape(B, C, R).transpose(0, 2, 1)
    x2 = jnp.pad(vmap3, ((0, 0), (0, RP - R), (0, 0))).reshape(B * RP, C)
    wa_t = W_a.T
    wb_t = W_b.T
    wv_t = jnp.pad(Wv, ((0, RP - R), (0, 0))).T
    wg_t = jnp.pad(Wg, ((0, RP - R), (0, 0))).T
    wh3 = jnp.pad(Wh[0], (0, RP - R)).reshape(1, 1, RP)
    wcat = jnp.concatenate([W_ih, W_hh], axis=1).T
    bg2 = (b_ih + b_hh).reshape(1, 4 * H)
    cap32 = captions.astype(jnp.int32)

    v3, vp3, vg = _encoder(x2, wa_t, b_a.reshape(1, H), wb_t,
                           b_b.reshape(1, H), wv_t)
    hid = _decode(v3, vp3, vg, wg_t, wh3, wcat, bg2, cap32, embed)
    xb = hid.transpose(1, 0, 2).reshape(B * T, 2 * H).astype(jnp.bfloat16)
    scores = _mlp(xb, W_mlp, b_mlp.reshape(1, VOCAB))
    return scores.reshape(B, T, VOCAB)


# speedup vs baseline: 1.0076x; 1.0076x over previous
import jax
import jax.numpy as jnp
from jax.experimental import pallas as pl
from jax.experimental.pallas import tpu as pltpu

B, T = 64, 20
C, R = 2048, 49
RP = 56
H, E, VOCAB = 512, 256, 32000
F32 = jnp.float32

_CN = (((1,), (1,)), ((), ()))

_EBB = 8
_EROWS = _EBB * RP


def _enc_body(x_ref, wa_ref, ba_ref, wb_ref, bb_ref, wv_ref,
              v_ref, vp_ref, vg_ref):
    xts = [jnp.pad(x_ref[b].T, ((0, RP - R), (0, 0))) for b in range(_EBB)]
    x = jnp.concatenate(xts, axis=0)
    v2 = jax.lax.dot_general(x, wa_ref[...], _CN,
                             preferred_element_type=F32) + ba_ref[...]
    v2 = jnp.maximum(v2, 0.0)
    m = jax.lax.broadcasted_iota(jnp.int32, (_EBB, _EROWS), 1)
    bidx = jax.lax.broadcasted_iota(jnp.int32, (_EBB, _EROWS), 0)
    sel = jnp.where(m // RP == bidx, 1.0 / R, 0.0).astype(F32)
    a_g = jnp.dot(sel, x, preferred_element_type=F32)
    vg = jax.lax.dot_general(a_g, wb_ref[...], _CN,
                             preferred_element_type=F32) + bb_ref[...]
    vg_ref[...] = jnp.maximum(vg, 0.0)
    vp2 = jax.lax.dot_general(v2, wv_ref[...], _CN,
                              preferred_element_type=F32)
    v_ref[...] = v2.reshape(_EBB, RP, H)
    vp_ref[...] = vp2.reshape(_EBB, RP, R)


def _encoder(feat3, w_a, ba2, w_b, bb2, wv):
    n = B // _EBB
    return pl.pallas_call(
        _enc_body,
        grid=(n,),
        in_specs=[
            pl.BlockSpec((_EBB, C, R), lambda i: (i, 0, 0)),
            pl.BlockSpec((H, C), lambda i: (0, 0)),
            pl.BlockSpec((1, H), lambda i: (0, 0)),
            pl.BlockSpec((H, C), lambda i: (0, 0)),
            pl.BlockSpec((1, H), lambda i: (0, 0)),
            pl.BlockSpec((R, H), lambda i: (0, 0)),
        ],
        out_specs=[
            pl.BlockSpec((_EBB, RP, H), lambda i: (i, 0, 0)),
            pl.BlockSpec((_EBB, RP, R), lambda i: (i, 0, 0)),
            pl.BlockSpec((_EBB, H), lambda i: (i, 0)),
        ],
        out_shape=[
            jax.ShapeDtypeStruct((B, RP, H), F32),
            jax.ShapeDtypeStruct((B, RP, R), F32),
            jax.ShapeDtypeStruct((B, H), F32),
        ],
        compiler_params=pltpu.CompilerParams(
            dimension_semantics=("parallel",),
            vmem_limit_bytes=100 * 1024 * 1024,
        ),
    )(feat3, w_a, ba2, w_b, bb2, wv)


_BH = B // 2


def _dec_body(v_ref, vp_ref, vg_ref, wg_ref, wh_ref, wih_ref, whh_ref,
              bg_ref, cap_ref, emb_hbm, hid_ref, ebuf, sem):
    b0 = pl.program_id(0) * _BH

    def fetch(t, slot):
        for b in range(_BH):
            idx = cap_ref[b0 + b, t]
            pltpu.make_async_copy(
                emb_hbm.at[pl.ds(idx, 1)],
                ebuf.at[slot, pl.ds(b, 1)],
                sem.at[slot],
            ).start()

    fetch(0, 0)
    h = vg_ref[...]
    c = h
    wh_v = wh_ref[...]
    rmask = jax.lax.broadcasted_iota(jnp.int32, (_BH, RP), 1) < R

    for t in range(T):
        slot = t % 2
        if t + 1 < T:
            fetch(t + 1, (t + 1) % 2)
        pltpu.make_async_copy(ebuf.at[slot], ebuf.at[slot], sem.at[slot]).wait()
        e_t = ebuf[slot]
        hwg = jax.lax.dot_general(h, wg_ref[...], _CN,
                                  preferred_element_type=F32)
        content = vp_ref[...] + hwg[:, None, :]
        z = jnp.sum(jnp.tanh(content) * wh_v, axis=2)
        z = jnp.where(rmask, z, -1e30)
        ez = jnp.exp(z - jnp.max(z, axis=1, keepdims=True))
        alpha = ez / jnp.sum(ez, axis=1, keepdims=True)
        c_att = jnp.sum(alpha[:, :, None] * v_ref[...], axis=1)
        xce = jnp.concatenate([c_att, e_t], axis=1)
        gates = (jax.lax.dot_general(xce, wih_ref[...], _CN,
                                     preferred_element_type=F32)
                 + jax.lax.dot_general(h, whh_ref[...], _CN,
                                       preferred_element_type=F32)
                 + bg_ref[...])
        ii = jax.nn.sigmoid(gates[:, 0:H])
        ff = jax.nn.sigmoid(gates[:, H:2 * H])
        gg = jnp.tanh(gates[:, 2 * H:3 * H])
        oo = jax.nn.sigmoid(gates[:, 3 * H:4 * H])
        c = ff * c + ii * gg
        h = oo * jnp.tanh(c)
        hid_ref[:, t, :] = jnp.concatenate([c_att, h], axis=1)


def _decode(v3, vp3, vg, wg, wh3, w_ih, w_hh, bg2, cap32, embed):
    return pl.pallas_call(
        _dec_body,
        grid=(2,),
        in_specs=[
            pl.BlockSpec((_BH, RP, H), lambda i: (i, 0, 0)),
            pl.BlockSpec((_BH, RP, R), lambda i: (i, 0, 0)),
            pl.BlockSpec((_BH, H), lambda i: (i, 0)),
            pl.BlockSpec((R, H), lambda i: (0, 0)),
            pl.BlockSpec((1, 1, R), lambda i: (0, 0, 0)),
            pl.BlockSpec((4 * H, H + E), lambda i: (0, 0)),
            pl.BlockSpec((4 * H, H), lambda i: (0, 0)),
            pl.BlockSpec((1, 4 * H), lambda i: (0, 0)),
            pl.BlockSpec(memory_space=pltpu.SMEM),
            pl.BlockSpec(memory_space=pl.ANY),
        ],
        out_specs=pl.BlockSpec((_BH, T, 2 * H), lambda i: (i, 0, 0)),
        out_shape=jax.ShapeDtypeStruct((B, T, 2 * H), F32),
        scratch_shapes=[
            pltpu.VMEM((2, _BH, E), F32),
            pltpu.SemaphoreType.DMA((2,)),
        ],
        compiler_params=pltpu.CompilerParams(
            dimension_semantics=("parallel",),
            vmem_limit_bytes=100 * 1024 * 1024,
        ),
    )(v3, vp3, vg, wg, wh3, w_ih, w_hh, bg2, cap32, embed)


_VB = 1280


def _mlp_body(x_ref, w_ref, b_ref, o_ref):
    x = x_ref[...].astype(jnp.bfloat16)
    w = w_ref[...].astype(jnp.bfloat16)
    s = jax.lax.dot_general(x, w, _CN, preferred_element_type=F32)
    o_ref[...] = s + b_ref[...]


def _mlp(x2, w_mlp, bm2):
    n = VOCAB // _VB
    return pl.pallas_call(
        _mlp_body,
        grid=(n,),
        in_specs=[
            pl.BlockSpec((B * T, 2 * H), lambda i: (0, 0)),
            pl.BlockSpec((_VB, 2 * H), lambda i: (i, 0)),
            pl.BlockSpec((1, _VB), lambda i: (0, i)),
        ],
        out_specs=pl.BlockSpec((B * T, _VB), lambda i: (0, i)),
        out_shape=jax.ShapeDtypeStruct((B * T, VOCAB), F32),
        compiler_params=pltpu.CompilerParams(
            dimension_semantics=("parallel",),
            vmem_limit_bytes=100 * 1024 * 1024,
        ),
    )(x2, w_mlp, bm2)


def kernel(feat, captions, lengths, W_a, b_a, W_b, b_b, embed,
           Wv, Wg, Wh, W_ih, W_hh, b_ih, b_hh, W_mlp, b_mlp):
    feat3 = feat.reshape(B, C, R)
    wh3 = Wh.reshape(1, 1, R)
    bg2 = (b_ih + b_hh).reshape(1, 4 * H)
    cap32 = captions.astype(jnp.int32)

    v3, vp3, vg = _encoder(feat3, W_a, b_a.reshape(1, H), W_b,
                           b_b.reshape(1, H), Wv)
    hid = _decode(v3, vp3, vg, Wg, wh3, W_ih, W_hh, bg2, cap32, embed)
    scores = _mlp(hid.reshape(B * T, 2 * H), W_mlp, b_mlp.reshape(1, VOCAB))
    return scores.reshape(B, T, VOCAB)


# speedup vs baseline: 1.4949x; 1.4836x over previous
import jax
import jax.numpy as jnp
from jax.experimental import pallas as pl
from jax.experimental.pallas import tpu as pltpu

B, T = 64, 20
C, R = 2048, 49
RP = 56
H, E, VOCAB = 512, 256, 32000
F32 = jnp.float32

_CN = (((1,), (1,)), ((), ()))

_EBB = 8
_EROWS = _EBB * RP


def _enc_body(x_ref, wa_ref, ba_ref, wb_ref, bb_ref, wv_ref,
              v_ref, vp_ref, vg_ref):
    xts = [jnp.pad(x_ref[b].T, ((0, RP - R), (0, 0))) for b in range(_EBB)]
    x = jnp.concatenate(xts, axis=0)
    v2 = jax.lax.dot_general(x, wa_ref[...], _CN,
                             preferred_element_type=F32) + ba_ref[...]
    v2 = jnp.maximum(v2, 0.0)
    m = jax.lax.broadcasted_iota(jnp.int32, (_EBB, _EROWS), 1)
    bidx = jax.lax.broadcasted_iota(jnp.int32, (_EBB, _EROWS), 0)
    sel = jnp.where(m // RP == bidx, 1.0 / R, 0.0).astype(F32)
    a_g = jnp.dot(sel, x, preferred_element_type=F32)
    vg = jax.lax.dot_general(a_g, wb_ref[...], _CN,
                             preferred_element_type=F32) + bb_ref[...]
    vg_ref[...] = jnp.maximum(vg, 0.0)
    vp2 = jax.lax.dot_general(v2, wv_ref[...], _CN,
                              preferred_element_type=F32)
    v_ref[...] = v2.reshape(_EBB, RP, H)
    vp_ref[...] = vp2.reshape(_EBB, RP, R)


def _encoder(feat3, w_a, ba2, w_b, bb2, wv):
    n = B // _EBB
    return pl.pallas_call(
        _enc_body,
        grid=(n,),
        in_specs=[
            pl.BlockSpec((_EBB, C, R), lambda i: (i, 0, 0)),
            pl.BlockSpec((H, C), lambda i: (0, 0)),
            pl.BlockSpec((1, H), lambda i: (0, 0)),
            pl.BlockSpec((H, C), lambda i: (0, 0)),
            pl.BlockSpec((1, H), lambda i: (0, 0)),
            pl.BlockSpec((R, H), lambda i: (0, 0)),
        ],
        out_specs=[
            pl.BlockSpec((_EBB, RP, H), lambda i: (i, 0, 0)),
            pl.BlockSpec((_EBB, RP, R), lambda i: (i, 0, 0)),
            pl.BlockSpec((_EBB, H), lambda i: (i, 0)),
        ],
        out_shape=[
            jax.ShapeDtypeStruct((B, RP, H), F32),
            jax.ShapeDtypeStruct((B, RP, R), F32),
            jax.ShapeDtypeStruct((B, H), F32),
        ],
        compiler_params=pltpu.CompilerParams(
            dimension_semantics=("parallel",),
            vmem_limit_bytes=100 * 1024 * 1024,
        ),
    )(feat3, w_a, ba2, w_b, bb2, wv)


_BH = B // 2


def _dec_body(v_ref, vp_ref, vg_ref, wg_ref, wh_ref, wih_ref, whh_ref,
              bg_ref, cap_ref, emb_hbm, hid_ref, ebuf, sem):
    b0 = pl.program_id(0) * _BH

    def fetch(t, slot):
        for b in range(_BH):
            idx = cap_ref[b0 + b, t]
            pltpu.make_async_copy(
                emb_hbm.at[pl.ds(idx, 1)],
                ebuf.at[slot, pl.ds(b, 1)],
                sem.at[slot],
            ).start()

    fetch(0, 0)
    h = vg_ref[...]
    c = h
    wh_v = wh_ref[...]
    rmask = jax.lax.broadcasted_iota(jnp.int32, (_BH, RP), 1) < R

    for t in range(T):
        slot = t % 2
        if t + 1 < T:
            fetch(t + 1, (t + 1) % 2)
        pltpu.make_async_copy(ebuf.at[slot], ebuf.at[slot], sem.at[slot]).wait()
        e_t = ebuf[slot]
        hwg = jax.lax.dot_general(h, wg_ref[...], _CN,
                                  preferred_element_type=F32)
        content = vp_ref[...] + hwg[:, None, :]
        z = jnp.sum(jnp.tanh(content) * wh_v, axis=2)
        z = jnp.where(rmask, z, -1e30)
        ez = jnp.exp(z - jnp.max(z, axis=1, keepdims=True))
        alpha = ez / jnp.sum(ez, axis=1, keepdims=True)
        c_att = jnp.sum(alpha[:, :, None] * v_ref[...], axis=1)
        xce = jnp.concatenate([c_att, e_t], axis=1)
        gates = (jax.lax.dot_general(xce, wih_ref[...], _CN,
                                     preferred_element_type=F32)
                 + jax.lax.dot_general(h, whh_ref[...], _CN,
                                       preferred_element_type=F32)
                 + bg_ref[...])
        ii = jax.nn.sigmoid(gates[:, 0:H])
        ff = jax.nn.sigmoid(gates[:, H:2 * H])
        gg = jnp.tanh(gates[:, 2 * H:3 * H])
        oo = jax.nn.sigmoid(gates[:, 3 * H:4 * H])
        c = ff * c + ii * gg
        h = oo * jnp.tanh(c)
        hid_ref[:, t, :] = jnp.concatenate([c_att, h], axis=1)


def _decode(v3, vp3, vg, wg, wh3, w_ih, w_hh, bg2, cap32, embed):
    return pl.pallas_call(
        _dec_body,
        grid=(2,),
        in_specs=[
            pl.BlockSpec((_BH, RP, H), lambda i: (i, 0, 0)),
            pl.BlockSpec((_BH, RP, R), lambda i: (i, 0, 0)),
            pl.BlockSpec((_BH, H), lambda i: (i, 0)),
            pl.BlockSpec((R, H), lambda i: (0, 0)),
            pl.BlockSpec((1, 1, R), lambda i: (0, 0, 0)),
            pl.BlockSpec((4 * H, H + E), lambda i: (0, 0)),
            pl.BlockSpec((4 * H, H), lambda i: (0, 0)),
            pl.BlockSpec((1, 4 * H), lambda i: (0, 0)),
            pl.BlockSpec(memory_space=pltpu.SMEM),
            pl.BlockSpec(memory_space=pl.ANY),
        ],
        out_specs=pl.BlockSpec((_BH, T, 2 * H), lambda i: (i, 0, 0)),
        out_shape=jax.ShapeDtypeStruct((B, T, 2 * H), F32),
        scratch_shapes=[
            pltpu.VMEM((2, _BH, E), F32),
            pltpu.SemaphoreType.DMA((2,)),
        ],
        compiler_params=pltpu.CompilerParams(
            dimension_semantics=("parallel",),
            vmem_limit_bytes=100 * 1024 * 1024,
        ),
    )(v3, vp3, vg, wg, wh3, w_ih, w_hh, bg2, cap32, embed)


_VB = 1280


def _mlp_body(x_ref, w_ref, b_ref, o_ref):
    x = x_ref[...].astype(jnp.bfloat16)
    w = w_ref[...].astype(jnp.bfloat16)
    s = jax.lax.dot_general(x, w, (((2,), (1,)), ((), ())),
                            preferred_element_type=F32)
    o_ref[...] = s + b_ref[...]


def _mlp(x3, w_mlp, bm3):
    n = VOCAB // _VB
    return pl.pallas_call(
        _mlp_body,
        grid=(n,),
        in_specs=[
            pl.BlockSpec((B, T, 2 * H), lambda i: (0, 0, 0)),
            pl.BlockSpec((_VB, 2 * H), lambda i: (i, 0)),
            pl.BlockSpec((1, 1, _VB), lambda i: (0, 0, i)),
        ],
        out_specs=pl.BlockSpec((B, T, _VB), lambda i: (0, 0, i)),
        out_shape=jax.ShapeDtypeStruct((B, T, VOCAB), F32),
        compiler_params=pltpu.CompilerParams(
            dimension_semantics=("parallel",),
            vmem_limit_bytes=100 * 1024 * 1024,
        ),
    )(x3, w_mlp, bm3)


def kernel(feat, captions, lengths, W_a, b_a, W_b, b_b, embed,
           Wv, Wg, Wh, W_ih, W_hh, b_ih, b_hh, W_mlp, b_mlp):
    feat3 = feat.reshape(B, C, R)
    wh3 = Wh.reshape(1, 1, R)
    bg2 = (b_ih + b_hh).reshape(1, 4 * H)
    cap32 = captions.astype(jnp.int32)

    v3, vp3, vg = _encoder(feat3, W_a, b_a.reshape(1, H), W_b,
                           b_b.reshape(1, H), Wv)
    hid = _decode(v3, vp3, vg, Wg, wh3, W_ih, W_hh, bg2, cap32, embed)
    return _mlp(hid, W_mlp, b_mlp.reshape(1, 1, VOCAB))


# speedup vs baseline: 2.7245x; 1.8225x over previous
import jax
import jax.numpy as jnp
from jax.experimental import pallas as pl
from jax.experimental.pallas import tpu as pltpu

B, T = 64, 20
C, R = 2048, 49
H, E, VOCAB = 512, 256, 32000
F32 = jnp.float32

_CL = (((1,), (1,)), ((), ()))

_EBB = 16
_EROWS = R * _EBB


def _enc_body(x_ref, wa_ref, ba_ref, wb_ref, bb_ref, wv_ref,
              v_ref, vp_ref, vg_ref):
    x = x_ref[...].reshape(_EROWS, C)
    v2 = jax.lax.dot_general(x, wa_ref[...], _CL,
                             preferred_element_type=F32) + ba_ref[...]
    v2 = jnp.maximum(v2, 0.0)
    m = jax.lax.broadcasted_iota(jnp.int32, (_EBB, _EROWS), 1)
    bidx = jax.lax.broadcasted_iota(jnp.int32, (_EBB, _EROWS), 0)
    sel = jnp.where(m % _EBB == bidx, 1.0 / R, 0.0).astype(F32)
    a_g = jnp.dot(sel, x, preferred_element_type=F32)
    vg = jax.lax.dot_general(a_g, wb_ref[...], _CL,
                             preferred_element_type=F32) + bb_ref[...]
    vg_ref[...] = jnp.maximum(vg, 0.0)
    vp2 = jax.lax.dot_general(v2, wv_ref[...], _CL,
                              preferred_element_type=F32)
    v_ref[...] = v2.reshape(R, _EBB, H)
    vp_ref[...] = vp2.reshape(R, _EBB, R)


def _encoder(xrb, w_a, ba2, w_b, bb2, wv):
    n = B // _EBB
    return pl.pallas_call(
        _enc_body,
        grid=(n,),
        in_specs=[
            pl.BlockSpec((R, _EBB, C), lambda i: (0, i, 0)),
            pl.BlockSpec((H, C), lambda i: (0, 0)),
            pl.BlockSpec((1, H), lambda i: (0, 0)),
            pl.BlockSpec((H, C), lambda i: (0, 0)),
            pl.BlockSpec((1, H), lambda i: (0, 0)),
            pl.BlockSpec((R, H), lambda i: (0, 0)),
        ],
        out_specs=[
            pl.BlockSpec((R, _EBB, H), lambda i: (0, i, 0)),
            pl.BlockSpec((R, _EBB, R), lambda i: (0, i, 0)),
            pl.BlockSpec((_EBB, H), lambda i: (i, 0)),
        ],
        out_shape=[
            jax.ShapeDtypeStruct((R, B, H), F32),
            jax.ShapeDtypeStruct((R, B, R), F32),
            jax.ShapeDtypeStruct((B, H), F32),
        ],
        compiler_params=pltpu.CompilerParams(
            dimension_semantics=("parallel",),
            vmem_limit_bytes=100 * 1024 * 1024,
        ),
    )(xrb, w_a, ba2, w_b, bb2, wv)


_BH = B // 2


def _dec_body(v_ref, vp_ref, vg_ref, wg_ref, wh_ref, wih_ref, whh_ref,
              bg_ref, cap_ref, emb_hbm, hid_ref, ebuf, sem):
    b0 = pl.program_id(0) * _BH

    def fetch(t, slot):
        for b in range(_BH):
            idx = cap_ref[b0 + b, t]
            pltpu.make_async_copy(
                emb_hbm.at[pl.ds(idx, 1)],
                ebuf.at[slot, pl.ds(b, 1)],
                sem.at[slot],
            ).start()

    fetch(0, 0)
    h = vg_ref[...]
    c = h
    wh_v = wh_ref[...]

    for t in range(T):
        slot = t % 2
        if t + 1 < T:
            fetch(t + 1, (t + 1) % 2)
        pltpu.make_async_copy(ebuf.at[slot], ebuf.at[slot], sem.at[slot]).wait()
        e_t = ebuf[slot]
        hwg = jax.lax.dot_general(h, wg_ref[...], _CL,
                                  preferred_element_type=F32)
        content = vp_ref[...] + hwg[None, :, :]
        z = jnp.sum(jnp.tanh(content) * wh_v, axis=2)
        ez = jnp.exp(z - jnp.max(z, axis=0, keepdims=True))
        alpha = ez / jnp.sum(ez, axis=0, keepdims=True)
        c_att = jnp.sum(alpha[:, :, None] * v_ref[...], axis=0)
        xce = jnp.concatenate([c_att, e_t], axis=1)
        gates = (jax.lax.dot_general(xce, wih_ref[...], _CL,
                                     preferred_element_type=F32)
                 + jax.lax.dot_general(h, whh_ref[...], _CL,
                                       preferred_element_type=F32)
                 + bg_ref[...])
        ii = jax.nn.sigmoid(gates[:, 0:H])
        ff = jax.nn.sigmoid(gates[:, H:2 * H])
        gg = jnp.tanh(gates[:, 2 * H:3 * H])
        oo = jax.nn.sigmoid(gates[:, 3 * H:4 * H])
        c = ff * c + ii * gg
        h = oo * jnp.tanh(c)
        hid_ref[t] = jnp.concatenate([c_att, h], axis=1)


def _decode(v3, vp3, vg, wg, wh3, w_ih, w_hh, bg2, cap32, embed):
    return pl.pallas_call(
        _dec_body,
        grid=(2,),
        in_specs=[
            pl.BlockSpec((R, _BH, H), lambda i: (0, i, 0)),
            pl.BlockSpec((R, _BH, R), lambda i: (0, i, 0)),
            pl.BlockSpec((_BH, H), lambda i: (i, 0)),
            pl.BlockSpec((R, H), lambda i: (0, 0)),
            pl.BlockSpec((1, 1, R), lambda i: (0, 0, 0)),
            pl.BlockSpec((4 * H, H + E), lambda i: (0, 0)),
            pl.BlockSpec((4 * H, H), lambda i: (0, 0)),
            pl.BlockSpec((1, 4 * H), lambda i: (0, 0)),
            pl.BlockSpec(memory_space=pltpu.SMEM),
            pl.BlockSpec(memory_space=pl.ANY),
        ],
        out_specs=pl.BlockSpec((T, _BH, 2 * H), lambda i: (0, i, 0)),
        out_shape=jax.ShapeDtypeStruct((T, B, 2 * H), F32),
        scratch_shapes=[
            pltpu.VMEM((2, _BH, E), F32),
            pltpu.SemaphoreType.DMA((2,)),
        ],
        compiler_params=pltpu.CompilerParams(
            dimension_semantics=("parallel",),
            vmem_limit_bytes=100 * 1024 * 1024,
        ),
    )(v3, vp3, vg, wg, wh3, w_ih, w_hh, bg2, cap32, embed)


_VB = 1280


def _mlp_body(x_ref, w_ref, b_ref, o_ref):
    x = x_ref[...].astype(jnp.bfloat16)
    w = w_ref[...].astype(jnp.bfloat16)
    s = jax.lax.dot_general(x, w, (((2,), (1,)), ((), ())),
                            preferred_element_type=F32)
    o_ref[...] = s + b_ref[...]


def _mlp(x3, w_mlp, bm3):
    n = VOCAB // _VB
    return pl.pallas_call(
        _mlp_body,
        grid=(n,),
        in_specs=[
            pl.BlockSpec((T, B, 2 * H), lambda i: (0, 0, 0)),
            pl.BlockSpec((_VB, 2 * H), lambda i: (i, 0)),
            pl.BlockSpec((1, 1, _VB), lambda i: (0, 0, i)),
        ],
        out_specs=pl.BlockSpec((T, B, _VB), lambda i: (0, 0, i)),
        out_shape=jax.ShapeDtypeStruct((T, B, VOCAB), F32),
        compiler_params=pltpu.CompilerParams(
            dimension_semantics=("parallel",),
            vmem_limit_bytes=100 * 1024 * 1024,
        ),
    )(x3, w_mlp, bm3)


def kernel(feat, captions, lengths, W_a, b_a, W_b, b_b, embed,
           Wv, Wg, Wh, W_ih, W_hh, b_ih, b_hh, W_mlp, b_mlp):
    xrb = feat.transpose(2, 3, 0, 1).reshape(R, B, C)
    wh3 = Wh.reshape(1, 1, R)
    bg2 = (b_ih + b_hh).reshape(1, 4 * H)
    cap32 = captions.astype(jnp.int32)

    v3, vp3, vg = _encoder(xrb, W_a, b_a.reshape(1, H), W_b,
                           b_b.reshape(1, H), Wv)
    hid = _decode(v3, vp3, vg, Wg, wh3, W_ih, W_hh, bg2, cap32, embed)
    stb = _mlp(hid, W_mlp, b_mlp.reshape(1, 1, VOCAB))
    return stb.transpose(1, 0, 2)


# speedup vs baseline: 2.8362x; 1.0410x over previous
import jax
import jax.numpy as jnp
from jax.experimental import pallas as pl
from jax.experimental.pallas import tpu as pltpu

B, T = 64, 20
C, R = 2048, 49
H, E, VOCAB = 512, 256, 32000
F32 = jnp.float32

_CL = (((1,), (1,)), ((), ()))

_EBB = 16
_EROWS = R * _EBB


def _enc_body(x_ref, wa_ref, ba_ref, wb_ref, bb_ref, wv_ref,
              v_ref, vp_ref, vg_ref):
    x = x_ref[...].reshape(_EROWS, C)
    v2 = jax.lax.dot_general(x, wa_ref[...], _CL,
                             preferred_element_type=F32) + ba_ref[...]
    v2 = jnp.maximum(v2, 0.0)
    m = jax.lax.broadcasted_iota(jnp.int32, (_EBB, _EROWS), 1)
    bidx = jax.lax.broadcasted_iota(jnp.int32, (_EBB, _EROWS), 0)
    sel = jnp.where(m % _EBB == bidx, 1.0 / R, 0.0).astype(F32)
    a_g = jnp.dot(sel, x, preferred_element_type=F32)
    vg = jax.lax.dot_general(a_g, wb_ref[...], _CL,
                             preferred_element_type=F32) + bb_ref[...]
    vg_ref[...] = jnp.maximum(vg, 0.0)
    vp2 = jax.lax.dot_general(v2, wv_ref[...], _CL,
                              preferred_element_type=F32)
    v_ref[...] = v2.reshape(R, _EBB, H)
    vp_ref[...] = vp2.reshape(R, _EBB, R)


def _encoder(xrb, w_a, ba2, w_b, bb2, wv):
    n = B // _EBB
    return pl.pallas_call(
        _enc_body,
        grid=(n,),
        in_specs=[
            pl.BlockSpec((R, _EBB, C), lambda i: (0, i, 0)),
            pl.BlockSpec((H, C), lambda i: (0, 0)),
            pl.BlockSpec((1, H), lambda i: (0, 0)),
            pl.BlockSpec((H, C), lambda i: (0, 0)),
            pl.BlockSpec((1, H), lambda i: (0, 0)),
            pl.BlockSpec((R, H), lambda i: (0, 0)),
        ],
        out_specs=[
            pl.BlockSpec((R, _EBB, H), lambda i: (0, i, 0)),
            pl.BlockSpec((R, _EBB, R), lambda i: (0, i, 0)),
            pl.BlockSpec((_EBB, H), lambda i: (i, 0)),
        ],
        out_shape=[
            jax.ShapeDtypeStruct((R, B, H), F32),
            jax.ShapeDtypeStruct((R, B, R), F32),
            jax.ShapeDtypeStruct((B, H), F32),
        ],
        compiler_params=pltpu.CompilerParams(
            dimension_semantics=("parallel",),
            vmem_limit_bytes=100 * 1024 * 1024,
        ),
    )(xrb, w_a, ba2, w_b, bb2, wv)


_BH = B


def _dec_body(v_ref, vp_ref, vg_ref, wg_ref, wh_ref, wih_ref, whh_ref,
              bg_ref, cap_ref, emb_hbm, hid_ref, ebuf, sem):
    b0 = pl.program_id(0) * _BH

    def fetch(t, slot):
        for b in range(_BH):
            idx = cap_ref[b0 + b, t]
            pltpu.make_async_copy(
                emb_hbm.at[pl.ds(idx, 1)],
                ebuf.at[slot, pl.ds(b, 1)],
                sem.at[slot],
            ).start()

    fetch(0, 0)
    h = vg_ref[...]
    c = h
    wh_v = wh_ref[...]

    for t in range(T):
        slot = t % 2
        if t + 1 < T:
            fetch(t + 1, (t + 1) % 2)
        pltpu.make_async_copy(ebuf.at[slot], ebuf.at[slot], sem.at[slot]).wait()
        e_t = ebuf[slot]
        hwg = jax.lax.dot_general(h, wg_ref[...], _CL,
                                  preferred_element_type=F32)
        content = vp_ref[...] + hwg[None, :, :]
        z = jnp.sum(jnp.tanh(content) * wh_v, axis=2)
        ez = jnp.exp(z - jnp.max(z, axis=0, keepdims=True))
        alpha = ez / jnp.sum(ez, axis=0, keepdims=True)
        c_att = jnp.sum(alpha[:, :, None] * v_ref[...], axis=0)
        xce = jnp.concatenate([c_att, e_t], axis=1)
        gates = (jax.lax.dot_general(xce, wih_ref[...], _CL,
                                     preferred_element_type=F32)
                 + jax.lax.dot_general(h, whh_ref[...], _CL,
                                       preferred_element_type=F32)
                 + bg_ref[...])
        ii = jax.nn.sigmoid(gates[:, 0:H])
        ff = jax.nn.sigmoid(gates[:, H:2 * H])
        gg = jnp.tanh(gates[:, 2 * H:3 * H])
        oo = jax.nn.sigmoid(gates[:, 3 * H:4 * H])
        c = ff * c + ii * gg
        h = oo * jnp.tanh(c)
        hid_ref[t] = jnp.concatenate([c_att, h], axis=1)


def _decode(v3, vp3, vg, wg, wh3, w_ih, w_hh, bg2, cap32, embed):
    return pl.pallas_call(
        _dec_body,
        grid=(1,),
        in_specs=[
            pl.BlockSpec((R, _BH, H), lambda i: (0, i, 0)),
            pl.BlockSpec((R, _BH, R), lambda i: (0, i, 0)),
            pl.BlockSpec((_BH, H), lambda i: (i, 0)),
            pl.BlockSpec((R, H), lambda i: (0, 0)),
            pl.BlockSpec((1, 1, R), lambda i: (0, 0, 0)),
            pl.BlockSpec((4 * H, H + E), lambda i: (0, 0)),
            pl.BlockSpec((4 * H, H), lambda i: (0, 0)),
            pl.BlockSpec((1, 4 * H), lambda i: (0, 0)),
            pl.BlockSpec(memory_space=pltpu.SMEM),
            pl.BlockSpec(memory_space=pl.ANY),
        ],
        out_specs=pl.BlockSpec((T, _BH, 2 * H), lambda i: (0, i, 0)),
        out_shape=jax.ShapeDtypeStruct((T, B, 2 * H), F32),
        scratch_shapes=[
            pltpu.VMEM((2, _BH, E), F32),
            pltpu.SemaphoreType.DMA((2,)),
        ],
        compiler_params=pltpu.CompilerParams(
            dimension_semantics=("parallel",),
            vmem_limit_bytes=100 * 1024 * 1024,
        ),
    )(v3, vp3, vg, wg, wh3, w_ih, w_hh, bg2, cap32, embed)


_VB = 640


def _mlp_body(x_ref, w_ref, b_ref, o_ref):
    x = x_ref[...].astype(jnp.bfloat16)
    w = w_ref[...].astype(jnp.bfloat16)
    s = jax.lax.dot_general(x, w, (((2,), (1,)), ((), ())),
                            preferred_element_type=F32)
    o_ref[...] = s + b_ref[...]


def _mlp(x3, w_mlp, bm3):
    n = VOCAB // _VB
    return pl.pallas_call(
        _mlp_body,
        grid=(n,),
        in_specs=[
            pl.BlockSpec((T, B, 2 * H), lambda i: (0, 0, 0)),
            pl.BlockSpec((_VB, 2 * H), lambda i: (i, 0)),
            pl.BlockSpec((1, 1, _VB), lambda i: (0, 0, i)),
        ],
        out_specs=pl.BlockSpec((T, B, _VB), lambda i: (0, 0, i)),
        out_shape=jax.ShapeDtypeStruct((T, B, VOCAB), F32),
        compiler_params=pltpu.CompilerParams(
            dimension_semantics=("parallel",),
            vmem_limit_bytes=100 * 1024 * 1024,
        ),
    )(x3, w_mlp, bm3)


def kernel(feat, captions, lengths, W_a, b_a, W_b, b_b, embed,
           Wv, Wg, Wh, W_ih, W_hh, b_ih, b_hh, W_mlp, b_mlp):
    xrb = feat.transpose(2, 3, 0, 1).reshape(R, B, C)
    wh3 = Wh.reshape(1, 1, R)
    bg2 = (b_ih + b_hh).reshape(1, 4 * H)
    cap32 = captions.astype(jnp.int32)

    v3, vp3, vg = _encoder(xrb, W_a, b_a.reshape(1, H), W_b,
                           b_b.reshape(1, H), Wv)
    hid = _decode(v3, vp3, vg, Wg, wh3, W_ih, W_hh, bg2, cap32, embed)
    stb = _mlp(hid, W_mlp, b_mlp.reshape(1, 1, VOCAB))
    return stb.transpose(1, 0, 2)


# speedup vs baseline: 3.1668x; 1.1165x over previous
import jax
import jax.numpy as jnp
from jax.experimental import pallas as pl
from jax.experimental.pallas import tpu as pltpu

B, T = 64, 20
C, R = 2048, 49
H, E, VOCAB = 512, 256, 32000
F32 = jnp.float32

_CL = (((1,), (1,)), ((), ()))

_EBB = 16
_EROWS = R * _EBB


def _enc_body(x_ref, wa_ref, ba_ref, wb_ref, bb_ref, wv_ref,
              v_ref, vp_ref, vg_ref):
    x = x_ref[...].reshape(_EROWS, C)
    v2 = jax.lax.dot_general(x, wa_ref[...], _CL,
                             preferred_element_type=F32) + ba_ref[...]
    v2 = jnp.maximum(v2, 0.0)
    m = jax.lax.broadcasted_iota(jnp.int32, (_EBB, _EROWS), 1)
    bidx = jax.lax.broadcasted_iota(jnp.int32, (_EBB, _EROWS), 0)
    sel = jnp.where(m % _EBB == bidx, 1.0 / R, 0.0).astype(F32)
    a_g = jnp.dot(sel, x, preferred_element_type=F32)
    vg = jax.lax.dot_general(a_g, wb_ref[...], _CL,
                             preferred_element_type=F32) + bb_ref[...]
    vg_ref[...] = jnp.maximum(vg, 0.0)
    vp2 = jax.lax.dot_general(v2, wv_ref[...], _CL,
                              preferred_element_type=F32)
    v_ref[...] = v2.reshape(R, _EBB, H)
    vp_ref[...] = vp2.reshape(R, _EBB, R)


def _encoder(xrb, w_a, ba2, w_b, bb2, wv):
    n = B // _EBB
    return pl.pallas_call(
        _enc_body,
        grid=(n,),
        in_specs=[
            pl.BlockSpec((R, _EBB, C), lambda i: (0, i, 0)),
            pl.BlockSpec((H, C), lambda i: (0, 0)),
            pl.BlockSpec((1, H), lambda i: (0, 0)),
            pl.BlockSpec((H, C), lambda i: (0, 0)),
            pl.BlockSpec((1, H), lambda i: (0, 0)),
            pl.BlockSpec((R, H), lambda i: (0, 0)),
        ],
        out_specs=[
            pl.BlockSpec((R, _EBB, H), lambda i: (0, i, 0)),
            pl.BlockSpec((R, _EBB, R), lambda i: (0, i, 0)),
            pl.BlockSpec((_EBB, H), lambda i: (i, 0)),
        ],
        out_shape=[
            jax.ShapeDtypeStruct((R, B, H), F32),
            jax.ShapeDtypeStruct((R, B, R), F32),
            jax.ShapeDtypeStruct((B, H), F32),
        ],
        compiler_params=pltpu.CompilerParams(
            dimension_semantics=("parallel",),
            vmem_limit_bytes=100 * 1024 * 1024,
        ),
    )(xrb, w_a, ba2, w_b, bb2, wv)


_BH = B


def _dec_body(v_ref, vp_ref, vg_ref, wg_ref, wh_ref, wih_ref, whh_ref,
              bg_ref, cap_ref, emb_hbm, hid_ref, ebuf, sem):
    b0 = pl.program_id(0) * _BH

    def fetch(t, slot):
        for b in range(_BH):
            idx = cap_ref[b0 + b, t]
            pltpu.make_async_copy(
                emb_hbm.at[pl.ds(idx, 1)],
                ebuf.at[slot, pl.ds(b, 1)],
                sem.at[slot],
            ).start()

    fetch(0, 0)
    fetch(1, 1)
    h = vg_ref[...]
    c = h
    wh_v = wh_ref[...]

    for t in range(T):
        slot = t % 3
        if t + 2 < T:
            fetch(t + 2, (t + 2) % 3)
        pltpu.make_async_copy(ebuf.at[slot], ebuf.at[slot], sem.at[slot]).wait()
        e_t = ebuf[slot]
        hwg = jax.lax.dot_general(h, wg_ref[...], _CL,
                                  preferred_element_type=F32)
        content = vp_ref[...] + hwg[None, :, :]
        z = jnp.sum(jnp.tanh(content) * wh_v, axis=2)
        ez = jnp.exp(z - jnp.max(z, axis=0, keepdims=True))
        alpha = ez / jnp.sum(ez, axis=0, keepdims=True)
        c_att = jnp.sum(alpha[:, :, None] * v_ref[...], axis=0)
        xce = jnp.concatenate([c_att, e_t], axis=1)
        gates = (jax.lax.dot_general(xce, wih_ref[...], _CL,
                                     preferred_element_type=F32)
                 + jax.lax.dot_general(h, whh_ref[...], _CL,
                                       preferred_element_type=F32)
                 + bg_ref[...])
        ii = jax.nn.sigmoid(gates[:, 0:H])
        ff = jax.nn.sigmoid(gates[:, H:2 * H])
        gg = jnp.tanh(gates[:, 2 * H:3 * H])
        oo = jax.nn.sigmoid(gates[:, 3 * H:4 * H])
        c = ff * c + ii * gg
        h = oo * jnp.tanh(c)
        hid_ref[t] = jnp.concatenate([c_att, h], axis=1)


def _decode(v3, vp3, vg, wg, wh3, w_ih, w_hh, bg2, cap32, embed):
    return pl.pallas_call(
        _dec_body,
        grid=(1,),
        in_specs=[
            pl.BlockSpec((R, _BH, H), lambda i: (0, i, 0)),
            pl.BlockSpec((R, _BH, R), lambda i: (0, i, 0)),
            pl.BlockSpec((_BH, H), lambda i: (i, 0)),
            pl.BlockSpec((R, H), lambda i: (0, 0)),
            pl.BlockSpec((1, 1, R), lambda i: (0, 0, 0)),
            pl.BlockSpec((4 * H, H + E), lambda i: (0, 0)),
            pl.BlockSpec((4 * H, H), lambda i: (0, 0)),
            pl.BlockSpec((1, 4 * H), lambda i: (0, 0)),
            pl.BlockSpec(memory_space=pltpu.SMEM),
            pl.BlockSpec(memory_space=pl.ANY),
        ],
        out_specs=pl.BlockSpec((T, _BH, 2 * H), lambda i: (0, i, 0)),
        out_shape=jax.ShapeDtypeStruct((T, B, 2 * H), F32),
        scratch_shapes=[
            pltpu.VMEM((3, _BH, E), F32),
            pltpu.SemaphoreType.DMA((3,)),
        ],
        compiler_params=pltpu.CompilerParams(
            dimension_semantics=("parallel",),
            vmem_limit_bytes=100 * 1024 * 1024,
        ),
    )(v3, vp3, vg, wg, wh3, w_ih, w_hh, bg2, cap32, embed)


_VB = 1280


def _mlp_body(x_ref, w_ref, b_ref, o_ref, xb_ref):
    @pl.when(pl.program_id(0) == 0)
    def _():
        xb_ref[...] = x_ref[...].astype(jnp.bfloat16)
    w = w_ref[...].astype(jnp.bfloat16)
    s = jax.lax.dot_general(xb_ref[...], w, (((2,), (1,)), ((), ())),
                            preferred_element_type=F32)
    o_ref[...] = s + b_ref[...]


def _mlp(x3, w_mlp, bm3):
    n = VOCAB // _VB
    return pl.pallas_call(
        _mlp_body,
        grid=(n,),
        in_specs=[
            pl.BlockSpec((T, B, 2 * H), lambda i: (0, 0, 0)),
            pl.BlockSpec((_VB, 2 * H), lambda i: (i, 0)),
            pl.BlockSpec((1, 1, _VB), lambda i: (0, 0, i)),
        ],
        out_specs=pl.BlockSpec((T, B, _VB), lambda i: (0, 0, i)),
        out_shape=jax.ShapeDtypeStruct((T, B, VOCAB), F32),
        scratch_shapes=[pltpu.VMEM((T, B, 2 * H), jnp.bfloat16)],
        compiler_params=pltpu.CompilerParams(
            dimension_semantics=("arbitrary",),
            vmem_limit_bytes=100 * 1024 * 1024,
        ),
    )(x3, w_mlp, bm3)


def kernel(feat, captions, lengths, W_a, b_a, W_b, b_b, embed,
           Wv, Wg, Wh, W_ih, W_hh, b_ih, b_hh, W_mlp, b_mlp):
    xrb = feat.transpose(2, 3, 0, 1).reshape(R, B, C)
    wh3 = Wh.reshape(1, 1, R)
    bg2 = (b_ih + b_hh).reshape(1, 4 * H)
    cap32 = captions.astype(jnp.int32)

    v3, vp3, vg = _encoder(xrb, W_a, b_a.reshape(1, H), W_b,
                           b_b.reshape(1, H), Wv)
    hid = _decode(v3, vp3, vg, Wg, wh3, W_ih, W_hh, bg2, cap32, embed)
    stb = _mlp(hid, W_mlp, b_mlp.reshape(1, 1, VOCAB))
    return stb.transpose(1, 0, 2)
